# baseline (speedup 1.0000x reference)
import functools

import jax
import jax.numpy as jnp
from jax import lax
from jax.experimental import pallas as pl
from jax.experimental.pallas import tpu as pltpu
from jax.experimental.pallas import tpu_sc as plsc

B = 4096
S = 200
D = 64
N_LOCS = 128

NC = 2
NS = 16
L = 16
NW = NC * NS
BPW = B // NW

SA = 128
SB = S - SA

_mesh = plsc.VectorSubcoreMesh(core_axis_name="c", subcore_axis_name="s")


@functools.partial(
    pl.kernel,
    mesh=_mesh,
    compiler_params=pltpu.CompilerParams(use_tc_tiling_on_sc=False),
    out_type=jax.ShapeDtypeStruct((B, D), jnp.float32),
    scratch_types=[
        pltpu.VMEM((BPW, SA), jnp.int32),
        pltpu.VMEM((BPW, SB), jnp.int32),
        pltpu.VMEM((SA, D), jnp.float32),
        pltpu.VMEM((SB, D), jnp.float32),
        pltpu.VMEM((BPW, D), jnp.float32),
        pltpu.SemaphoreType.DMA,
        pltpu.SemaphoreType.DMA,
    ],
)
def _pool_sc(xa_hbm, xb_hbm, emb_hbm, out_hbm,
             idx_a, idx_b, buf_a, buf_b, out_v, sem_a, sem_b):
    wid = lax.axis_index("s") * NC + lax.axis_index("c")
    base = wid * BPW

    pltpu.sync_copy(xa_hbm.at[pl.ds(base, BPW)], idx_a)
    pltpu.sync_copy(xb_hbm.at[pl.ds(base, BPW)], idx_b)

    def row(i, carry):
        cp_a = pltpu.async_copy(emb_hbm.at[idx_a.at[i]], buf_a, sem_a)
        cp_b = pltpu.async_copy(emb_hbm.at[idx_b.at[i]], buf_b, sem_b)
        cp_a.wait()
        cp_b.wait()

        def red_a(j, ms):
            return tuple(
                jnp.maximum(ms[c], buf_a[j, pl.ds(c * L, L)]) for c in range(4)
            )

        def red_b(j, ms):
            return tuple(
                jnp.maximum(ms[c], buf_b[j, pl.ds(c * L, L)]) for c in range(4)
            )

        ms = tuple(jnp.full((L,), -jnp.inf, jnp.float32) for _ in range(4))
        ms = lax.fori_loop(0, SA, red_a, ms)
        ms = lax.fori_loop(0, SB, red_b, ms)
        for c in range(4):
            out_v[i, pl.ds(c * L, L)] = ms[c]
        return carry

    lax.fori_loop(0, BPW, row, 0)
    pltpu.sync_copy(out_v, out_hbm.at[pl.ds(base, BPW)])


def _mm_body(p_ref, w_ref, b_ref, o_ref):
    o_ref[...] = (
        lax.dot_general(
            p_ref[...], w_ref[...],
            (((1,), (1,)), ((), ())),
            preferred_element_type=jnp.float32,
        )
        + b_ref[...]
    )


_mm = pl.pallas_call(
    _mm_body,
    grid=(8,),
    in_specs=[
        pl.BlockSpec((B // 8, D), lambda i: (i, 0)),
        pl.BlockSpec((N_LOCS, D), lambda i: (0, 0)),
        pl.BlockSpec((1, N_LOCS), lambda i: (0, 0)),
    ],
    out_specs=pl.BlockSpec((B // 8, N_LOCS), lambda i: (i, 0)),
    out_shape=jax.ShapeDtypeStruct((B, N_LOCS), jnp.float32),
)


def kernel(x, emb, W_fc, b_fc):
    x = x.astype(jnp.int32)
    xa = x[:, :SA]
    xb = x[:, SA:]
    pooled = _pool_sc(xa, xb, emb)
    return _mm(pooled, W_fc, b_fc.reshape(1, N_LOCS))


# speedup vs baseline: 1.1571x; 1.1571x over previous
import functools

import jax
import jax.numpy as jnp
from jax import lax
from jax.experimental import pallas as pl
from jax.experimental.pallas import tpu as pltpu
from jax.experimental.pallas import tpu_sc as plsc

B = 4096
S = 200
D = 64
N_LOCS = 128

NC = 2
NS = 16
L = 16
NW = NC * NS
BPW = B // NW

SA = 128
SB = S - SA

_mesh = plsc.VectorSubcoreMesh(core_axis_name="c", subcore_axis_name="s")


@functools.partial(
    pl.kernel,
    mesh=_mesh,
    compiler_params=pltpu.CompilerParams(use_tc_tiling_on_sc=False),
    out_type=jax.ShapeDtypeStruct((B, D), jnp.float32),
    scratch_types=[
        pltpu.VMEM((BPW, SA), jnp.int32),
        pltpu.VMEM((BPW, SB), jnp.int32),
        pltpu.VMEM((SA, D), jnp.float32),
        pltpu.VMEM((SB, D), jnp.float32),
        pltpu.VMEM((SA, D), jnp.float32),
        pltpu.VMEM((SB, D), jnp.float32),
        pltpu.VMEM((BPW, D), jnp.float32),
        pltpu.SemaphoreType.DMA,
        pltpu.SemaphoreType.DMA,
        pltpu.SemaphoreType.DMA,
        pltpu.SemaphoreType.DMA,
    ],
)
def _pool_sc(xa_hbm, xb_hbm, emb_hbm, out_hbm,
             idx_a, idx_b, buf_a0, buf_b0, buf_a1, buf_b1, out_v,
             sem_a0, sem_b0, sem_a1, sem_b1):
    wid = lax.axis_index("s") * NC + lax.axis_index("c")
    base = wid * BPW

    pltpu.sync_copy(xa_hbm.at[pl.ds(base, BPW)], idx_a)
    pltpu.sync_copy(xb_hbm.at[pl.ds(base, BPW)], idx_b)

    def issue(i, buf_a, buf_b, sem_a, sem_b):
        pltpu.async_copy(emb_hbm.at[idx_a.at[i]], buf_a, sem_a)
        pltpu.async_copy(emb_hbm.at[idx_b.at[i]], buf_b, sem_b)

    def wait(buf_a, buf_b, sem_a, sem_b):
        pltpu.make_async_copy(emb_hbm.at[idx_a.at[0]], buf_a, sem_a).wait()
        pltpu.make_async_copy(emb_hbm.at[idx_b.at[0]], buf_b, sem_b).wait()

    def reduce_row(i, buf_a, buf_b):
        def red(buf):
            def body(j, ms):
                return tuple(
                    jnp.maximum(ms[c], buf[j, pl.ds(c * L, L)])
                    for c in range(4)
                )
            return body

        ms = tuple(jnp.full((L,), -jnp.inf, jnp.float32) for _ in range(4))
        ms = lax.fori_loop(0, SA, red(buf_a), ms, unroll=8)
        ms = lax.fori_loop(0, SB, red(buf_b), ms, unroll=8)
        for c in range(4):
            out_v[i, pl.ds(c * L, L)] = ms[c]

    issue(0, buf_a0, buf_b0, sem_a0, sem_b0)

    def pair(p, carry):
        i0 = 2 * p
        i1 = i0 + 1
        issue(i1, buf_a1, buf_b1, sem_a1, sem_b1)
        wait(buf_a0, buf_b0, sem_a0, sem_b0)
        reduce_row(i0, buf_a0, buf_b0)
        issue(jnp.minimum(i0 + 2, BPW - 1), buf_a0, buf_b0, sem_a0, sem_b0)
        wait(buf_a1, buf_b1, sem_a1, sem_b1)
        reduce_row(i1, buf_a1, buf_b1)
        return carry

    lax.fori_loop(0, BPW // 2, pair, 0)
    wait(buf_a0, buf_b0, sem_a0, sem_b0)
    pltpu.sync_copy(out_v, out_hbm.at[pl.ds(base, BPW)])


def _mm_body(p_ref, w_ref, b_ref, o_ref):
    o_ref[...] = (
        lax.dot_general(
            p_ref[...], w_ref[...],
            (((1,), (1,)), ((), ())),
            preferred_element_type=jnp.float32,
        )
        + b_ref[...]
    )


_mm = pl.pallas_call(
    _mm_body,
    grid=(8,),
    in_specs=[
        pl.BlockSpec((B // 8, D), lambda i: (i, 0)),
        pl.BlockSpec((N_LOCS, D), lambda i: (0, 0)),
        pl.BlockSpec((1, N_LOCS), lambda i: (0, 0)),
    ],
    out_specs=pl.BlockSpec((B // 8, N_LOCS), lambda i: (i, 0)),
    out_shape=jax.ShapeDtypeStruct((B, N_LOCS), jnp.float32),
)


def kernel(x, emb, W_fc, b_fc):
    x = x.astype(jnp.int32)
    xa = x[:, :SA]
    xb = x[:, SA:]
    pooled = _pool_sc(xa, xb, emb)
    return _mm(pooled, W_fc, b_fc.reshape(1, N_LOCS))


# speedup vs baseline: 1.2157x; 1.0507x over previous
import functools

import jax
import jax.numpy as jnp
from jax import lax
from jax.experimental import pallas as pl
from jax.experimental.pallas import tpu as pltpu
from jax.experimental.pallas import tpu_sc as plsc

B = 4096
S = 200
D = 64
N_LOCS = 128

NC = 2
NS = 16
L = 16
NW = NC * NS
BPW = B // NW

SA = 128
SB = S - SA

_mesh = plsc.VectorSubcoreMesh(core_axis_name="c", subcore_axis_name="s")


@functools.partial(
    pl.kernel,
    mesh=_mesh,
    compiler_params=pltpu.CompilerParams(use_tc_tiling_on_sc=False),
    out_type=jax.ShapeDtypeStruct((B, D), jnp.float32),
    scratch_types=[
        pltpu.VMEM((BPW, SA), jnp.int32),
        pltpu.VMEM((BPW, SB), jnp.int32),
        [pltpu.VMEM((SA, D), jnp.float32)] * 4,
        [pltpu.VMEM((SB, D), jnp.float32)] * 4,
        pltpu.VMEM((BPW, D), jnp.float32),
        [pltpu.SemaphoreType.DMA] * 4,
        [pltpu.SemaphoreType.DMA] * 4,
    ],
)
def _pool_sc(xa_hbm, xb_hbm, emb_hbm, out_hbm,
             idx_a, idx_b, bufs_a, bufs_b, out_v, sems_a, sems_b):
    NSLOT = 4
    wid = lax.axis_index("s") * NC + lax.axis_index("c")
    base = wid * BPW

    pltpu.sync_copy(xa_hbm.at[pl.ds(base, BPW)], idx_a)
    pltpu.sync_copy(xb_hbm.at[pl.ds(base, BPW)], idx_b)

    def issue(i, s):
        pltpu.async_copy(emb_hbm.at[idx_a.at[i]], bufs_a[s], sems_a[s])
        pltpu.async_copy(emb_hbm.at[idx_b.at[i]], bufs_b[s], sems_b[s])

    def wait(s):
        pltpu.make_async_copy(emb_hbm.at[idx_a.at[0]], bufs_a[s], sems_a[s]).wait()
        pltpu.make_async_copy(emb_hbm.at[idx_b.at[0]], bufs_b[s], sems_b[s]).wait()

    def reduce_row(i, s):
        def red(buf):
            def body(j, ms):
                return tuple(
                    jnp.maximum(ms[c], buf[j, pl.ds(c * L, L)])
                    for c in range(4)
                )
            return body

        ms = tuple(jnp.full((L,), -jnp.inf, jnp.float32) for _ in range(4))
        ms = lax.fori_loop(0, SA, red(bufs_a[s]), ms, unroll=8)
        ms = lax.fori_loop(0, SB, red(bufs_b[s]), ms, unroll=8)
        for c in range(4):
            out_v[i, pl.ds(c * L, L)] = ms[c]

    for s in range(NSLOT - 1):
        issue(s, s)

    def group(p, carry):
        i0 = NSLOT * p
        for s in range(NSLOT):
            i = i0 + s
            issue(jnp.minimum(i + NSLOT - 1, BPW - 1), (s + NSLOT - 1) % NSLOT)
            wait(s)
            reduce_row(i, s)
        return carry

    lax.fori_loop(0, BPW // NSLOT, group, 0)
    for s in range(NSLOT - 1):
        wait(s)
    pltpu.sync_copy(out_v, out_hbm.at[pl.ds(base, BPW)])


def _mm_body(p_ref, w_ref, b_ref, o_ref):
    o_ref[...] = (
        lax.dot_general(
            p_ref[...], w_ref[...],
            (((1,), (1,)), ((), ())),
            preferred_element_type=jnp.float32,
        )
        + b_ref[...]
    )


_mm = pl.pallas_call(
    _mm_body,
    grid=(8,),
    in_specs=[
        pl.BlockSpec((B // 8, D), lambda i: (i, 0)),
        pl.BlockSpec((N_LOCS, D), lambda i: (0, 0)),
        pl.BlockSpec((1, N_LOCS), lambda i: (0, 0)),
    ],
    out_specs=pl.BlockSpec((B // 8, N_LOCS), lambda i: (i, 0)),
    out_shape=jax.ShapeDtypeStruct((B, N_LOCS), jnp.float32),
)


def kernel(x, emb, W_fc, b_fc):
    x = x.astype(jnp.int32)
    xa = x[:, :SA]
    xb = x[:, SA:]
    pooled = _pool_sc(xa, xb, emb)
    return _mm(pooled, W_fc, b_fc.reshape(1, N_LOCS))
